# baseline (speedup 1.0000x reference)
import functools

import jax
import jax.numpy as jnp
from jax import lax
from jax.experimental import pallas as pl
from jax.experimental.pallas import tpu as pltpu
from jax.experimental.pallas import tpu_sc as plsc

NC = 2
NS = 16
NW = NC * NS
CH = 128


def _gather_body(bpw, nch, d,
                 user_table, item_table, uidx_hbm, iidx_hbm,
                 uout, iout,
                 uidx_v, iidx_v, urows, irows, sem):
  wid = lax.axis_index("s") * NC + lax.axis_index("c")
  base = wid * bpw
  pltpu.sync_copy(uidx_hbm.at[pl.ds(wid * nch, nch)], uidx_v)
  pltpu.sync_copy(iidx_hbm.at[pl.ds(wid * nch, nch)], iidx_v)
  copies = []
  for j in range(nch):
    copies.append(pltpu.async_copy(
        user_table.at[uidx_v.at[j]], urows.at[pl.ds(j * CH, CH)], sem))
    copies.append(pltpu.async_copy(
        item_table.at[iidx_v.at[j]], irows.at[pl.ds(j * CH, CH)], sem))
  for c in copies:
    c.wait()
  pltpu.sync_copy(urows, uout.at[pl.ds(base, bpw)])
  pltpu.sync_copy(irows, iout.at[pl.ds(base, bpw)])


def _sc_gather(user_table, item_table, uidx2d, iidx2d, b, d):
  bpw = b // NW
  nch = bpw // CH
  mesh = plsc.VectorSubcoreMesh(
      core_axis_name="c", subcore_axis_name="s",
      num_cores=NC, num_subcores=NS)
  f = pl.kernel(
      functools.partial(_gather_body, bpw, nch, d),
      out_type=(jax.ShapeDtypeStruct((b, d), jnp.float32),
                jax.ShapeDtypeStruct((b, d), jnp.float32)),
      mesh=mesh,
      scratch_types=[
          pltpu.VMEM((nch, CH), jnp.int32),
          pltpu.VMEM((nch, CH), jnp.int32),
          pltpu.VMEM((bpw, d), jnp.float32),
          pltpu.VMEM((bpw, d), jnp.float32),
          pltpu.SemaphoreType.DMA,
      ],
      compiler_params=pltpu.CompilerParams(use_tc_tiling_on_sc=False),
  )
  return f(user_table, item_table, uidx2d, iidx2d)


def _mlp_body(xu_ref, xi_ref, w1u_ref, w1i_ref, b1_ref, w2t_ref, b2_ref,
              w3_ref, b3_ref, o_ref):
  h = jnp.dot(xu_ref[...], w1u_ref[...], preferred_element_type=jnp.float32)
  h = h + jnp.dot(xi_ref[...], w1i_ref[...],
                  preferred_element_type=jnp.float32)
  h = jnp.maximum(h + b1_ref[...], 0.0)
  h = jnp.dot(h, w2t_ref[...], preferred_element_type=jnp.float32)
  h = jnp.maximum(h + b2_ref[...], 0.0)
  o_ref[...] = jnp.sum(h * w3_ref[...], axis=1) + b3_ref[0]


def _mlp(uemb, iemb, W1, b1, W2, b2, W3, b3, b):
  d = uemb.shape[1]
  w1u = W1[:, :d].T
  w1i = W1[:, d:].T
  w2t = W2.T
  bm = 2048
  grid = (b // bm,)
  return pl.pallas_call(
      _mlp_body,
      grid=grid,
      in_specs=[
          pl.BlockSpec((bm, d), lambda i: (i, 0)),
          pl.BlockSpec((bm, d), lambda i: (i, 0)),
          pl.BlockSpec(w1u.shape, lambda i: (0, 0)),
          pl.BlockSpec(w1i.shape, lambda i: (0, 0)),
          pl.BlockSpec((1, W1.shape[0]), lambda i: (0, 0)),
          pl.BlockSpec(w2t.shape, lambda i: (0, 0)),
          pl.BlockSpec((1, W2.shape[0]), lambda i: (0, 0)),
          pl.BlockSpec(W3.shape, lambda i: (0, 0)),
          pl.BlockSpec(memory_space=pltpu.SMEM),
      ],
      out_specs=pl.BlockSpec((bm,), lambda i: (i,)),
      out_shape=jax.ShapeDtypeStruct((b,), jnp.float32),
  )(uemb, iemb, w1u, w1i, b1.reshape(1, -1), w2t, b2.reshape(1, -1),
    W3, b3)


def kernel(user_ids, item_ids, user_table, item_table, W1, b1, W2, b2, W3, b3):
  b = user_ids.shape[0]
  d = user_table.shape[1]
  uidx2d = user_ids.astype(jnp.int32).reshape(b // CH, CH)
  iidx2d = item_ids.astype(jnp.int32).reshape(b // CH, CH)
  uemb, iemb = _sc_gather(user_table, item_table, uidx2d, iidx2d, b, d)
  return _mlp(uemb, iemb, W1, b1, W2, b2, W3, b3, b)


# speedup vs baseline: 1.3747x; 1.3747x over previous
import functools

import jax
import jax.numpy as jnp
from jax import lax
from jax.experimental import pallas as pl
from jax.experimental.pallas import tpu as pltpu
from jax.experimental.pallas import tpu_sc as plsc

NC = 2
NS = 16
NW = NC * NS
L = 16
RT = 256
RANGE = 31232
CHUNK_END = 999936
RING = 64


def _splat(x):
  return jnp.full((L,), x, jnp.int32)


def _gather_table(b, d, iota, wid, tab_t, tail, idx_hbm, scratch,
                  ids_v, hits_v, ch_v, chunks_v, ring_v, tail_v,
                  csem0, csem1, rsem):
  lo = wid * RANGE
  is_last = wid == NW - 1
  hi = jnp.where(is_last, tab_t.shape[1], lo + RANGE)
  nchunks = jnp.where(is_last, (CHUNK_END - (NW - 1) * RANGE) // RT,
                      RANGE // RT)

  pltpu.sync_copy(idx_hbm, ids_v)

  def scan(s, off):
    iv = ids_v[pl.ds(s * L, L)]
    m = jnp.logical_and(iv >= lo, iv < hi)
    val = lax.shift_left(iv - lo, 14) | ((s * L) + iota)
    plsc.store_compressed(hits_v.at[pl.ds(off, L)], val, mask=m)
    return off + jnp.sum(m.astype(jnp.int32))

  n_hits = lax.fori_loop(0, b // L, scan, 0)

  def issue_chunk(c, buf, sem):
    pltpu.async_copy(
        tab_t.at[pl.ds(0, d), pl.ds(lo + c * RT, RT)], buf, sem)

  def wait_chunk(buf, sem):
    pltpu.make_async_copy(
        tab_t.at[pl.ds(0, d), pl.ds(0, RT)], buf, sem).wait()

  def process(src_ref, sel_lo, sel_hi, col_major):
    def cscan(s, coff):
      base = lax.shift_left(s, 4)
      hv = hits_v[pl.ds(base, L)]
      rr = lax.shift_right_logical(hv, 14)
      m = jnp.logical_and(rr >= sel_lo, rr < sel_hi)
      m = jnp.logical_and(m, (base + iota) < n_hits)
      plsc.store_compressed(ch_v.at[pl.ds(coff, L)], hv, mask=m)
      return coff + jnp.sum(m.astype(jnp.int32))

    n_ch = lax.fori_loop(0, lax.shift_right_logical(n_hits + L - 1, 4),
                         cscan, 0)

    def hproc(t, carry):
      slot = t & (RING - 1)

      @pl.when(jnp.logical_and(t > 0, slot == 0))
      def _():
        def dr(q, cc):
          pltpu.make_async_copy(
              ring_v.at[pl.ds(0, 1)], scratch.at[pl.ds(0, 1)], rsem).wait()
          return cc
        lax.fori_loop(0, RING, dr, 0)

      grp = lax.shift_left(lax.shift_right_logical(t, 4), 4)
      hv16 = ch_v[pl.ds(grp, L)]
      lane = t - grp
      hval = jnp.max(jnp.where(iota == lane, hv16, 0))
      r_loc = lax.shift_right_logical(hval, 14) - sel_lo
      j = hval & ((1 << 14) - 1)
      for k in range(d // L):
        cv = iota + k * L
        if col_major:
          v = plsc.load_gather(src_ref, [cv, _splat(r_loc)])
        else:
          v = plsc.load_gather(src_ref, [_splat(r_loc), cv])
        ring_v[slot, pl.ds(k * L, L)] = v
      pltpu.async_copy(
          ring_v.at[pl.ds(slot, 1)], scratch.at[pl.ds(j, 1)], rsem)
      return carry

    lax.fori_loop(0, n_ch, hproc, 0)

    drained = lax.shift_left(
        lax.shift_right_logical(jnp.maximum(n_ch - 1, 0), 6), 6)

    def dr2(q, cc):
      pltpu.make_async_copy(
          ring_v.at[pl.ds(0, 1)], scratch.at[pl.ds(0, 1)], rsem).wait()
      return cc

    lax.fori_loop(0, n_ch - drained, dr2, 0)

  issue_chunk(0, chunks_v.at[0], csem0)
  issue_chunk(1, chunks_v.at[1], csem1)

  def pair(p, carry):
    c0 = lax.shift_left(p, 1)
    for bsel, sem in ((0, csem0), (1, csem1)):
      c = c0 + bsel
      buf = chunks_v.at[bsel]
      wait_chunk(buf, sem)
      process(buf, c * RT, (c + 1) * RT, True)

      @pl.when(c + 2 < nchunks)
      def _():
        issue_chunk(c + 2, buf, sem)
    return carry

  lax.fori_loop(0, lax.shift_right_logical(nchunks, 1), pair, 0)

  @pl.when(is_last)
  def _():
    pltpu.sync_copy(tail, tail_v)
    process(tail_v, CHUNK_END - lo, hi - lo, False)


def _gather_body(b, d, user_t, item_t, utail, itail, uidx, iidx,
                 uout, iout,
                 ids_v, hits_v, ch_v, chunks_v, ring_v, tail_v,
                 uscr, iscr, csem0, csem1, rsem):
  wid = lax.axis_index("s") * NC + lax.axis_index("c")
  iota = lax.iota(jnp.int32, L)
  for tab_t, tail, idx_hbm, scratch in (
      (user_t, utail, uidx, uscr), (item_t, itail, iidx, iscr)):
    _gather_table(b, d, iota, wid, tab_t, tail, idx_hbm, scratch,
                  ids_v, hits_v, ch_v, chunks_v, ring_v, tail_v,
                  csem0, csem1, rsem)
  base = wid * (b // NW)
  pltpu.sync_copy(uscr.at[pl.ds(base, b // NW)],
                  uout.at[pl.ds(base, b // NW)])
  pltpu.sync_copy(iscr.at[pl.ds(base, b // NW)],
                  iout.at[pl.ds(base, b // NW)])


def _sc_gather(user_t, item_t, utail, itail, uidx, iidx, b, d):
  mesh = plsc.VectorSubcoreMesh(
      core_axis_name="c", subcore_axis_name="s",
      num_cores=NC, num_subcores=NS)
  f = pl.kernel(
      functools.partial(_gather_body, b, d),
      out_type=(jax.ShapeDtypeStruct((b, d), jnp.float32),
                jax.ShapeDtypeStruct((b, d), jnp.float32)),
      mesh=mesh,
      scratch_types=[
          pltpu.VMEM((b,), jnp.int32),
          pltpu.VMEM((b + L,), jnp.int32),
          pltpu.VMEM((b + L,), jnp.int32),
          pltpu.VMEM((2, d, RT), jnp.float32),
          pltpu.VMEM((RING, d), jnp.float32),
          pltpu.VMEM((64, d), jnp.float32),
          pltpu.HBM((b, d), jnp.float32),
          pltpu.HBM((b, d), jnp.float32),
          pltpu.SemaphoreType.DMA,
          pltpu.SemaphoreType.DMA,
          pltpu.SemaphoreType.DMA,
      ],
      compiler_params=pltpu.CompilerParams(needs_layout_passes=False),
  )
  return f(user_t, item_t, utail, itail, uidx, iidx)


def _mlp_body(xu_ref, xi_ref, w1u_ref, w1i_ref, b1_ref, w2t_ref, b2_ref,
              w3_ref, b3_ref, o_ref):
  h = jnp.dot(xu_ref[...], w1u_ref[...], preferred_element_type=jnp.float32)
  h = h + jnp.dot(xi_ref[...], w1i_ref[...],
                  preferred_element_type=jnp.float32)
  h = jnp.maximum(h + b1_ref[...], 0.0)
  h = jnp.dot(h, w2t_ref[...], preferred_element_type=jnp.float32)
  h = jnp.maximum(h + b2_ref[...], 0.0)
  o_ref[...] = jnp.sum(h * w3_ref[...], axis=1) + b3_ref[0]


def _mlp(uemb, iemb, W1, b1, W2, b2, W3, b3, b):
  d = uemb.shape[1]
  w1u = W1[:, :d].T
  w1i = W1[:, d:].T
  w2t = W2.T
  bm = 2048
  grid = (b // bm,)
  return pl.pallas_call(
      _mlp_body,
      grid=grid,
      in_specs=[
          pl.BlockSpec((bm, d), lambda i: (i, 0)),
          pl.BlockSpec((bm, d), lambda i: (i, 0)),
          pl.BlockSpec(w1u.shape, lambda i: (0, 0)),
          pl.BlockSpec(w1i.shape, lambda i: (0, 0)),
          pl.BlockSpec((1, W1.shape[0]), lambda i: (0, 0)),
          pl.BlockSpec(w2t.shape, lambda i: (0, 0)),
          pl.BlockSpec((1, W2.shape[0]), lambda i: (0, 0)),
          pl.BlockSpec(W3.shape, lambda i: (0, 0)),
          pl.BlockSpec(memory_space=pltpu.SMEM),
      ],
      out_specs=pl.BlockSpec((bm,), lambda i: (i,)),
      out_shape=jax.ShapeDtypeStruct((b,), jnp.float32),
  )(uemb, iemb, w1u, w1i, b1.reshape(1, -1), w2t, b2.reshape(1, -1),
    W3, b3)


def kernel(user_ids, item_ids, user_table, item_table, W1, b1, W2, b2, W3, b3):
  b = user_ids.shape[0]
  d = user_table.shape[1]
  n = user_table.shape[0]
  uidx = user_ids.astype(jnp.int32)
  iidx = item_ids.astype(jnp.int32)
  user_t = user_table.T
  item_t = item_table.T
  utail = lax.slice(user_table, (CHUNK_END, 0), (n, d))
  itail = lax.slice(item_table, (CHUNK_END, 0), (n, d))
  uemb, iemb = _sc_gather(user_t, item_t, utail, itail, uidx, iidx, b, d)
  return _mlp(uemb, iemb, W1, b1, W2, b2, W3, b3, b)


# speedup vs baseline: 1.3791x; 1.0032x over previous
import functools

import jax
import jax.numpy as jnp
from jax import lax
from jax.experimental import pallas as pl
from jax.experimental.pallas import tpu as pltpu
from jax.experimental.pallas import tpu_sc as plsc

NC = 2
NS = 16
NW = NC * NS
L = 16
RT = 256
RANGE = 31232
CHUNK_END = 999936
RING = 64


def _splat(x):
  return jnp.full((L,), x, jnp.int32)


def _gather_table(b, d, iota, wid, tab_t, tail, idx_hbm, scratch,
                  ids_v, hits_v, ch_v, chunks_v, ring_v, tail_v,
                  csem0, csem1, rsem):
  lo = wid * RANGE
  is_last = wid == NW - 1
  hi = jnp.where(is_last, tab_t.shape[1], lo + RANGE)
  nchunks = jnp.where(is_last, (CHUNK_END - (NW - 1) * RANGE) // RT,
                      RANGE // RT)

  pltpu.sync_copy(idx_hbm, ids_v)

  def scan(s, off):
    iv = ids_v[pl.ds(s * L, L)]
    m = jnp.logical_and(iv >= lo, iv < hi)
    val = lax.shift_left(iv - lo, 14) | ((s * L) + iota)
    plsc.store_compressed(hits_v.at[pl.ds(off, L)], val, mask=m)
    return off + jnp.sum(m.astype(jnp.int32))

  n_hits = lax.fori_loop(0, b // L, scan, 0)

  def issue_chunk(c, buf, sem):
    pltpu.async_copy(
        tab_t.at[pl.ds(0, d), pl.ds(lo + c * RT, RT)], buf, sem)

  def wait_chunk(buf, sem):
    pltpu.make_async_copy(
        tab_t.at[pl.ds(0, d), pl.ds(0, RT)], buf, sem).wait()

  def process(src_ref, sel_lo, sel_hi, col_major, w0):
    def cscan(s, coff):
      base = lax.shift_left(s, 4)
      hv = hits_v[pl.ds(base, L)]
      rr = lax.shift_right_logical(hv, 14)
      m = jnp.logical_and(rr >= sel_lo, rr < sel_hi)
      m = jnp.logical_and(m, (base + iota) < n_hits)
      plsc.store_compressed(ch_v.at[pl.ds(coff, L)], hv, mask=m)
      return coff + jnp.sum(m.astype(jnp.int32))

    n_ch = lax.fori_loop(0, lax.shift_right_logical(n_hits + L - 1, 4),
                         cscan, 0)

    def hproc(t, w):
      slot = w & (RING - 1)

      @pl.when(jnp.logical_and(w > 0, slot == 0))
      def _():
        def dr(q, cc):
          pltpu.make_async_copy(
              ring_v.at[pl.ds(0, 1)], scratch.at[pl.ds(0, 1)], rsem).wait()
          return cc
        lax.fori_loop(0, RING, dr, 0)

      grp = lax.shift_left(lax.shift_right_logical(t, 4), 4)
      hv16 = ch_v[pl.ds(grp, L)]
      lane = t - grp
      hval = jnp.max(jnp.where(iota == lane, hv16, 0))
      r_loc = lax.shift_right_logical(hval, 14) - sel_lo
      j = hval & ((1 << 14) - 1)
      for k in range(d // L):
        cv = iota + k * L
        if col_major:
          v = plsc.load_gather(src_ref, [cv, _splat(r_loc)])
        else:
          v = plsc.load_gather(src_ref, [_splat(r_loc), cv])
        ring_v[slot, pl.ds(k * L, L)] = v
      pltpu.async_copy(
          ring_v.at[pl.ds(slot, 1)], scratch.at[pl.ds(j, 1)], rsem)
      return w + 1

    return lax.fori_loop(0, n_ch, hproc, w0)

  issue_chunk(0, chunks_v.at[0], csem0)
  issue_chunk(1, chunks_v.at[1], csem1)

  def pair(p, w):
    c0 = lax.shift_left(p, 1)
    for bsel, sem in ((0, csem0), (1, csem1)):
      c = c0 + bsel
      buf = chunks_v.at[bsel]
      wait_chunk(buf, sem)
      w = process(buf, c * RT, (c + 1) * RT, True, w)

      @pl.when(c + 2 < nchunks)
      def _():
        issue_chunk(c + 2, buf, sem)
    return w

  w = lax.fori_loop(0, lax.shift_right_logical(nchunks, 1), pair, 0)

  def do_tail():
    pltpu.sync_copy(tail, tail_v)
    return process(tail_v, CHUNK_END - lo, hi - lo, False, w)

  w = lax.cond(is_last, do_tail, lambda: w)

  drained = lax.shift_left(
      lax.shift_right_logical(jnp.maximum(w - 1, 0), 6), 6)

  def dr2(q, cc):
    pltpu.make_async_copy(
        ring_v.at[pl.ds(0, 1)], scratch.at[pl.ds(0, 1)], rsem).wait()
    return cc

  lax.fori_loop(0, w - drained, dr2, 0)


def _gather_body(b, d, user_t, item_t, utail, itail, uidx, iidx,
                 uout, iout,
                 ids_v, hits_v, ch_v, chunks_v, ring_v, tail_v,
                 uscr, iscr, csem0, csem1, rsem):
  wid = lax.axis_index("s") * NC + lax.axis_index("c")
  iota = lax.iota(jnp.int32, L)
  for tab_t, tail, idx_hbm, scratch in (
      (user_t, utail, uidx, uscr), (item_t, itail, iidx, iscr)):
    _gather_table(b, d, iota, wid, tab_t, tail, idx_hbm, scratch,
                  ids_v, hits_v, ch_v, chunks_v, ring_v, tail_v,
                  csem0, csem1, rsem)
  base = wid * (b // NW)
  pltpu.sync_copy(uscr.at[pl.ds(base, b // NW)],
                  uout.at[pl.ds(base, b // NW)])
  pltpu.sync_copy(iscr.at[pl.ds(base, b // NW)],
                  iout.at[pl.ds(base, b // NW)])


def _sc_gather(user_t, item_t, utail, itail, uidx, iidx, b, d):
  mesh = plsc.VectorSubcoreMesh(
      core_axis_name="c", subcore_axis_name="s",
      num_cores=NC, num_subcores=NS)
  f = pl.kernel(
      functools.partial(_gather_body, b, d),
      out_type=(jax.ShapeDtypeStruct((b, d), jnp.float32),
                jax.ShapeDtypeStruct((b, d), jnp.float32)),
      mesh=mesh,
      scratch_types=[
          pltpu.VMEM((b,), jnp.int32),
          pltpu.VMEM((b + L,), jnp.int32),
          pltpu.VMEM((b + L,), jnp.int32),
          pltpu.VMEM((2, d, RT), jnp.float32),
          pltpu.VMEM((RING, d), jnp.float32),
          pltpu.VMEM((64, d), jnp.float32),
          pltpu.HBM((b, d), jnp.float32),
          pltpu.HBM((b, d), jnp.float32),
          pltpu.SemaphoreType.DMA,
          pltpu.SemaphoreType.DMA,
          pltpu.SemaphoreType.DMA,
      ],
      compiler_params=pltpu.CompilerParams(needs_layout_passes=False),
  )
  return f(user_t, item_t, utail, itail, uidx, iidx)


def _mlp_body(xu_ref, xi_ref, w1u_ref, w1i_ref, b1_ref, w2t_ref, b2_ref,
              w3_ref, b3_ref, o_ref):
  h = jnp.dot(xu_ref[...], w1u_ref[...], preferred_element_type=jnp.float32)
  h = h + jnp.dot(xi_ref[...], w1i_ref[...],
                  preferred_element_type=jnp.float32)
  h = jnp.maximum(h + b1_ref[...], 0.0)
  h = jnp.dot(h, w2t_ref[...], preferred_element_type=jnp.float32)
  h = jnp.maximum(h + b2_ref[...], 0.0)
  o_ref[...] = jnp.sum(h * w3_ref[...], axis=1) + b3_ref[0]


def _mlp(uemb, iemb, W1, b1, W2, b2, W3, b3, b):
  d = uemb.shape[1]
  w1u = W1[:, :d].T
  w1i = W1[:, d:].T
  w2t = W2.T
  bm = 2048
  grid = (b // bm,)
  return pl.pallas_call(
      _mlp_body,
      grid=grid,
      in_specs=[
          pl.BlockSpec((bm, d), lambda i: (i, 0)),
          pl.BlockSpec((bm, d), lambda i: (i, 0)),
          pl.BlockSpec(w1u.shape, lambda i: (0, 0)),
          pl.BlockSpec(w1i.shape, lambda i: (0, 0)),
          pl.BlockSpec((1, W1.shape[0]), lambda i: (0, 0)),
          pl.BlockSpec(w2t.shape, lambda i: (0, 0)),
          pl.BlockSpec((1, W2.shape[0]), lambda i: (0, 0)),
          pl.BlockSpec(W3.shape, lambda i: (0, 0)),
          pl.BlockSpec(memory_space=pltpu.SMEM),
      ],
      out_specs=pl.BlockSpec((bm,), lambda i: (i,)),
      out_shape=jax.ShapeDtypeStruct((b,), jnp.float32),
  )(uemb, iemb, w1u, w1i, b1.reshape(1, -1), w2t, b2.reshape(1, -1),
    W3, b3)


def kernel(user_ids, item_ids, user_table, item_table, W1, b1, W2, b2, W3, b3):
  b = user_ids.shape[0]
  d = user_table.shape[1]
  n = user_table.shape[0]
  uidx = user_ids.astype(jnp.int32)
  iidx = item_ids.astype(jnp.int32)
  user_t = user_table.T
  item_t = item_table.T
  utail = lax.slice(user_table, (CHUNK_END, 0), (n, d))
  itail = lax.slice(item_table, (CHUNK_END, 0), (n, d))
  uemb, iemb = _sc_gather(user_t, item_t, utail, itail, uidx, iidx, b, d)
  return _mlp(uemb, iemb, W1, b1, W2, b2, W3, b3, b)


# speedup vs baseline: 1.8559x; 1.3457x over previous
import functools

import jax
import jax.numpy as jnp
from jax import lax
from jax.experimental import pallas as pl
from jax.experimental.pallas import tpu as pltpu
from jax.experimental.pallas import tpu_sc as plsc

NC = 2
NS = 16
NW = NC * NS
L = 16
K = 64
SUB = 8


def _gather_body(bpw, d, user3d, item3d, uidx_hbm, iidx_hbm, uout, iout,
                 idx_v, sv_v, slab_v, packed_v, sem):
  wid = lax.axis_index("s") * NC + lax.axis_index("c")
  nchk = bpw // K
  iota = lax.iota(jnp.int32, L)
  for table3d, idx_hbm, out_hbm in ((user3d, uidx_hbm, uout),
                                    (item3d, iidx_hbm, iout)):
    pltpu.sync_copy(idx_hbm.at[pl.ds(wid * nchk, nchk)], idx_v)
    for ch in range(nchk):
      for i in range(K // L):
        iv = idx_v[ch, pl.ds(i * L, L)]
        gv = lax.shift_right_logical(iv, 3)
        sv_v[pl.ds(i * L, L)] = lax.bitwise_and(iv, 7)

        def issue(l, carry, gv=gv, base=i * L):
          g = jnp.max(jnp.where(iota == l, gv, 0))
          pltpu.async_copy(
              table3d.at[g], slab_v.at[pl.ds((base + l) * SUB, SUB)], sem)
          return carry

        lax.fori_loop(0, L, issue, 0)

      def drain(l, carry):
        pltpu.make_async_copy(
            table3d.at[0], slab_v.at[pl.ds(0, SUB)], sem).wait()
        return carry

      lax.fori_loop(0, K, drain, 0)

      for j0 in range(0, K, L):
        jv = iota + j0
        sv = sv_v[pl.ds(j0, L)]
        rowv = jv * SUB + sv
        pjv = jv

        def body(c, carry, rowv=rowv, pjv=pjv):
          cv = jnp.full((L,), c, jnp.int32)
          v = plsc.load_gather(slab_v, [rowv, cv])
          plsc.store_scatter(packed_v, [pjv, cv], v)
          return carry

        lax.fori_loop(0, d, body, 0)
      pltpu.sync_copy(packed_v, out_hbm.at[pl.ds(wid * bpw + ch * K, K)])


def _sc_gather(user3d, item3d, uidx2d, iidx2d, b, d):
  bpw = b // NW
  nchk = bpw // K
  mesh = plsc.VectorSubcoreMesh(
      core_axis_name="c", subcore_axis_name="s",
      num_cores=NC, num_subcores=NS)
  f = pl.kernel(
      functools.partial(_gather_body, bpw, d),
      out_type=(jax.ShapeDtypeStruct((b, d), jnp.float32),
                jax.ShapeDtypeStruct((b, d), jnp.float32)),
      mesh=mesh,
      scratch_types=[
          pltpu.VMEM((nchk, K), jnp.int32),
          pltpu.VMEM((K,), jnp.int32),
          pltpu.VMEM((K * SUB, d), jnp.float32),
          pltpu.VMEM((K, d), jnp.float32),
          pltpu.SemaphoreType.DMA,
      ],
      compiler_params=pltpu.CompilerParams(needs_layout_passes=False),
  )
  return f(user3d, item3d, uidx2d, iidx2d)


def _mlp_body(xu_ref, xi_ref, w1u_ref, w1i_ref, b1_ref, w2t_ref, b2_ref,
              w3_ref, b3_ref, o_ref):
  h = jnp.dot(xu_ref[...], w1u_ref[...], preferred_element_type=jnp.float32)
  h = h + jnp.dot(xi_ref[...], w1i_ref[...],
                  preferred_element_type=jnp.float32)
  h = jnp.maximum(h + b1_ref[...], 0.0)
  h = jnp.dot(h, w2t_ref[...], preferred_element_type=jnp.float32)
  h = jnp.maximum(h + b2_ref[...], 0.0)
  o_ref[...] = jnp.sum(h * w3_ref[...], axis=1) + b3_ref[0]


def _mlp(uemb, iemb, W1, b1, W2, b2, W3, b3, b):
  d = uemb.shape[1]
  w1u = W1[:, :d].T
  w1i = W1[:, d:].T
  w2t = W2.T
  bm = 2048
  grid = (b // bm,)
  return pl.pallas_call(
      _mlp_body,
      grid=grid,
      in_specs=[
          pl.BlockSpec((bm, d), lambda i: (i, 0)),
          pl.BlockSpec((bm, d), lambda i: (i, 0)),
          pl.BlockSpec(w1u.shape, lambda i: (0, 0)),
          pl.BlockSpec(w1i.shape, lambda i: (0, 0)),
          pl.BlockSpec((1, W1.shape[0]), lambda i: (0, 0)),
          pl.BlockSpec(w2t.shape, lambda i: (0, 0)),
          pl.BlockSpec((1, W2.shape[0]), lambda i: (0, 0)),
          pl.BlockSpec(W3.shape, lambda i: (0, 0)),
          pl.BlockSpec(memory_space=pltpu.SMEM),
      ],
      out_specs=pl.BlockSpec((bm,), lambda i: (i,)),
      out_shape=jax.ShapeDtypeStruct((b,), jnp.float32),
  )(uemb, iemb, w1u, w1i, b1.reshape(1, -1), w2t, b2.reshape(1, -1),
    W3, b3)


def kernel(user_ids, item_ids, user_table, item_table, W1, b1, W2, b2, W3, b3):
  b = user_ids.shape[0]
  d = user_table.shape[1]
  n = user_table.shape[0]
  uidx2d = user_ids.astype(jnp.int32).reshape(b // K, K)
  iidx2d = item_ids.astype(jnp.int32).reshape(b // K, K)
  user3d = user_table.reshape(n // SUB, SUB, d)
  item3d = item_table.reshape(n // SUB, SUB, d)
  uemb, iemb = _sc_gather(user3d, item3d, uidx2d, iidx2d, b, d)
  return _mlp(uemb, iemb, W1, b1, W2, b2, W3, b3, b)
